# baseline (speedup 1.0000x reference)
import jax
import jax.numpy as jnp
from jax import lax
from jax.experimental import pallas as pl
from jax.experimental.pallas import tpu as pltpu
from jax.experimental.pallas import tpu_sc as plsc

_NC = 2
_NS = 16
_L = 16
_NW = _NC * _NS
_B = 16384
_BPW = _B // _NW
_CHUNK = 128
_NCHUNK = _BPW // _CHUNK
_D = 32
_PACK = 128 // _D


def _mf_body(user_hbm, item_hbm, u_hbm, v_hbm, out_hbm,
             idx_u, idx_v, idxr_u, idxr_v, urows, vrows, out_buf, sem):
    wid = lax.axis_index("s") * _NC + lax.axis_index("c")

    pltpu.sync_copy(user_hbm.at[wid], idx_u)
    pltpu.sync_copy(item_hbm.at[wid], idx_v)

    iota = lax.iota(jnp.int32, _L)

    for h in range(2):
        for cc in range(2):
            c = 2 * h + cc
            for k in range(_CHUNK // _L):
                sl = pl.ds(k * _L, _L)
                idxr_u[cc, sl] = lax.shift_right_logical(idx_u[c, sl], 2)
        copies = []
        for cc in range(2):
            c = 2 * h + cc
            copies.append(pltpu.async_copy(
                u_hbm.at[idxr_u.at[cc]], urows.at[pl.ds(cc * _CHUNK, _CHUNK)], sem))
            copies.append(pltpu.async_copy(
                v_hbm.at[idx_v.at[c]], vrows.at[pl.ds(cc * _CHUNK, _CHUNK)], sem))
        for cp in copies:
            cp.wait()

        for cc in range(2):
            c = 2 * h + cc
            for k in range(_CHUNK // _L):
                sl = pl.ds(k * _L, _L)
                brow = cc * _CHUNK + k * _L + iota
                ucol = lax.shift_left(jnp.bitwise_and(idx_u[c, sl], _PACK - 1), 5)
                acc = jnp.zeros((_L,), jnp.float32)
                for j in range(_D):
                    jv = jnp.full((_L,), j, jnp.int32)
                    uu = plsc.load_gather(urows, [brow, ucol + j])
                    vv = plsc.load_gather(vrows, [brow, jv])
                    acc = acc + uu * vv
                out_buf[pl.ds(c * _CHUNK + k * _L, _L)] = acc

    pltpu.sync_copy(out_buf, out_hbm.at[pl.ds(wid * _BPW, _BPW)])


def kernel(user, item, U, V):
    user3 = user.reshape(_NW, _NCHUNK, _CHUNK)
    item3 = item.reshape(_NW, _NCHUNK, _CHUNK)
    u_packed = lax.optimization_barrier(U.reshape(U.shape[0] // _PACK, 128))
    v_padded = lax.optimization_barrier(jnp.pad(V, ((0, 0), (0, 128 - _D))))
    mesh = plsc.VectorSubcoreMesh(core_axis_name="c", subcore_axis_name="s")
    fn = pl.kernel(
        _mf_body,
        mesh=mesh,
        out_type=jax.ShapeDtypeStruct((_B,), jnp.float32),
        compiler_params=pltpu.CompilerParams(
            needs_layout_passes=False, use_tc_tiling_on_sc=False),
        scratch_types=[
            pltpu.VMEM((_NCHUNK, _CHUNK), jnp.int32),
            pltpu.VMEM((_NCHUNK, _CHUNK), jnp.int32),
            pltpu.VMEM((2, _CHUNK), jnp.int32),
            pltpu.VMEM((2, _CHUNK), jnp.int32),
            pltpu.VMEM((2 * _CHUNK, 128), jnp.float32),
            pltpu.VMEM((2 * _CHUNK, 128), jnp.float32),
            pltpu.VMEM((_BPW,), jnp.float32),
            pltpu.SemaphoreType.DMA,
        ],
    )
    return fn(user3, item3, u_packed, v_padded)


# speedup vs baseline: 1.0072x; 1.0072x over previous
import jax
import jax.numpy as jnp
from jax import lax
from jax.experimental import pallas as pl
from jax.experimental.pallas import tpu as pltpu
from jax.experimental.pallas import tpu_sc as plsc

_NC = 2
_NS = 16
_L = 16
_NW = _NC * _NS
_B = 16384
_BPW = _B // _NW
_CHUNK = 128
_NCHUNK = _BPW // _CHUNK
_D = 32
_VW = 128


def _mf_body(user_hbm, item_hbm, u_hbm, v_hbm, out_hbm,
             idx_u, idx_v, urows, vrows, out_buf, sem):
    wid = lax.axis_index("s") * _NC + lax.axis_index("c")

    pltpu.sync_copy(user_hbm.at[wid], idx_u)
    pltpu.sync_copy(item_hbm.at[wid], idx_v)

    copies = []
    for c in range(_NCHUNK):
        copies.append(pltpu.async_copy(
            u_hbm.at[idx_u.at[c]], urows.at[pl.ds(c * _CHUNK, _CHUNK)], sem))
        copies.append(pltpu.async_copy(
            v_hbm.at[idx_v.at[c]], vrows.at[pl.ds(c * _CHUNK, _CHUNK)], sem))
    for cp in copies:
        cp.wait()

    iota = lax.iota(jnp.int32, _L)

    def body(g, carry):
        b_idx = g * _L + iota
        acc = jnp.zeros((_L,), jnp.float32)
        for j in range(_D):
            jv = jnp.full((_L,), j, jnp.int32)
            uu = plsc.load_gather(urows, [b_idx, jv])
            vv = plsc.load_gather(vrows, [b_idx, jv])
            acc = acc + uu * vv
        out_buf[pl.ds(g * _L, _L)] = acc
        return carry

    lax.fori_loop(0, _BPW // _L, body, 0)

    pltpu.sync_copy(out_buf, out_hbm.at[pl.ds(wid * _BPW, _BPW)])


def kernel(user, item, U, V):
    user3 = user.reshape(_NW, _NCHUNK, _CHUNK)
    item3 = item.reshape(_NW, _NCHUNK, _CHUNK)
    v_padded = lax.optimization_barrier(jnp.pad(V, ((0, 0), (0, _VW - _D))))
    mesh = plsc.VectorSubcoreMesh(core_axis_name="c", subcore_axis_name="s")
    fn = pl.kernel(
        _mf_body,
        mesh=mesh,
        out_type=jax.ShapeDtypeStruct((_B,), jnp.float32),
        compiler_params=pltpu.CompilerParams(
            needs_layout_passes=False, use_tc_tiling_on_sc=False),
        scratch_types=[
            pltpu.VMEM((_NCHUNK, _CHUNK), jnp.int32),
            pltpu.VMEM((_NCHUNK, _CHUNK), jnp.int32),
            pltpu.VMEM((_BPW, _D), jnp.float32),
            pltpu.VMEM((_BPW, _VW), jnp.float32),
            pltpu.VMEM((_BPW,), jnp.float32),
            pltpu.SemaphoreType.DMA,
        ],
    )
    return fn(user3, item3, U, v_padded)


# speedup vs baseline: 1.0312x; 1.0238x over previous
import jax
import jax.numpy as jnp
from jax import lax
from jax.experimental import pallas as pl
from jax.experimental.pallas import tpu as pltpu
from jax.experimental.pallas import tpu_sc as plsc

_NC = 2
_NS = 16
_L = 16
_NW = _NC * _NS
_B = 16384
_BPW = _B // _NW
_CHUNK = 128
_NCHUNK = _BPW // _CHUNK
_D = 32


def _mf_body(user_hbm, item_hbm, u_hbm, v_hbm, out_hbm,
             idx_u, idx_v, rows_u, rows_v, out_buf, sem):
    wid = lax.axis_index("s") * _NC + lax.axis_index("c")

    pltpu.sync_copy(user_hbm.at[wid], idx_u)
    pltpu.sync_copy(item_hbm.at[wid], idx_v)

    copies = []
    for c in range(_NCHUNK):
        copies.append(pltpu.async_copy(
            u_hbm.at[idx_u.at[c]], rows_u.at[pl.ds(c * _CHUNK, _CHUNK)], sem))
        copies.append(pltpu.async_copy(
            v_hbm.at[idx_v.at[c]], rows_v.at[pl.ds(c * _CHUNK, _CHUNK)], sem))
    for cp in copies:
        cp.wait()

    iota = lax.iota(jnp.int32, _L)

    def body(g, carry):
        b_idx = g * _L + iota
        acc = jnp.zeros((_L,), jnp.float32)
        for j in range(_D):
            jv = jnp.full((_L,), j, jnp.int32)
            uu = plsc.load_gather(rows_u, [b_idx, jv])
            vv = plsc.load_gather(rows_v, [b_idx, jv])
            acc = acc + uu * vv
        out_buf[pl.ds(g * _L, _L)] = acc
        return carry

    lax.fori_loop(0, _BPW // _L, body, 0)

    pltpu.sync_copy(out_buf, out_hbm.at[pl.ds(wid * _BPW, _BPW)])


def kernel(user, item, U, V):
    user3 = user.reshape(_NW, _NCHUNK, _CHUNK)
    item3 = item.reshape(_NW, _NCHUNK, _CHUNK)
    mesh = plsc.VectorSubcoreMesh(core_axis_name="c", subcore_axis_name="s")
    fn = pl.kernel(
        _mf_body,
        mesh=mesh,
        out_type=jax.ShapeDtypeStruct((_B,), jnp.float32),
        compiler_params=pltpu.CompilerParams(
            needs_layout_passes=False, use_tc_tiling_on_sc=False),
        scratch_types=[
            pltpu.VMEM((_NCHUNK, _CHUNK), jnp.int32),
            pltpu.VMEM((_NCHUNK, _CHUNK), jnp.int32),
            pltpu.VMEM((_BPW, _D), jnp.float32),
            pltpu.VMEM((_BPW, _D), jnp.float32),
            pltpu.VMEM((_BPW,), jnp.float32),
            pltpu.SemaphoreType.DMA,
        ],
    )
    return fn(user3, item3, U, V)
